# baseline (speedup 1.0000x reference)
import functools

import jax
import jax.numpy as jnp
from jax import lax
from jax.experimental import pallas as pl
from jax.experimental.pallas import tpu as pltpu
from jax.experimental.pallas import tpu_sc as plsc


def _sc_gather(movie_table, user_table, mids, uids):
    B = mids.shape[0]
    EMB = movie_table.shape[1]
    info = plsc.get_sparse_core_info()
    nc, ns = info.num_cores, info.num_subcores
    nw = nc * ns
    b_per_w = B // nw
    mesh = plsc.VectorSubcoreMesh(core_axis_name="c", subcore_axis_name="s")

    @functools.partial(
        pl.kernel,
        mesh=mesh,
        compiler_params=pltpu.CompilerParams(use_tc_tiling_on_sc=False),
        out_type=[
            jax.ShapeDtypeStruct((B, EMB), jnp.float32),
            jax.ShapeDtypeStruct((B, EMB), jnp.float32),
        ],
        scratch_types=[
            pltpu.VMEM((b_per_w,), jnp.int32),
            pltpu.VMEM((b_per_w, EMB), jnp.float32),
            pltpu.VMEM((b_per_w,), jnp.int32),
            pltpu.VMEM((b_per_w, EMB), jnp.float32),
            pltpu.SemaphoreType.DMA,
            pltpu.SemaphoreType.DMA,
        ],
    )
    def gather_k(mtab, utab, mid, uid, mout, uout,
                 midx_v, mrows_v, uidx_v, urows_v, msem, usem):
        wid = lax.axis_index("s") * nc + lax.axis_index("c")
        base = wid * b_per_w
        pltpu.sync_copy(mid.at[pl.ds(base, b_per_w)], midx_v)
        pltpu.sync_copy(uid.at[pl.ds(base, b_per_w)], uidx_v)
        mcp = pltpu.async_copy(mtab.at[midx_v], mrows_v, msem)
        ucp = pltpu.async_copy(utab.at[uidx_v], urows_v, usem)
        mcp.wait()
        ucp.wait()
        pltpu.sync_copy(mrows_v, mout.at[pl.ds(base, b_per_w)])
        pltpu.sync_copy(urows_v, uout.at[pl.ds(base, b_per_w)])

    return gather_k(movie_table, user_table, mids, uids)


def _mlp_body(mv, us, gn, vt, wg, bg, w1m, w1u, w1g, w1v, b1,
              w2, b2, w3, b3, w4, b4, out):
    f32 = jnp.float32
    g = jnp.dot(gn[...], wg[...], preferred_element_type=f32, precision=jax.lax.Precision.HIGHEST) + bg[...]
    g = jnp.maximum(g, 0.0)
    x = (jnp.dot(mv[...], w1m[...], preferred_element_type=f32, precision=jax.lax.Precision.HIGHEST)
         + jnp.dot(us[...], w1u[...], preferred_element_type=f32, precision=jax.lax.Precision.HIGHEST)
         + jnp.dot(g, w1g[...], preferred_element_type=f32, precision=jax.lax.Precision.HIGHEST)
         + vt[...] * w1v[...]
         + b1[...])
    x = jnp.maximum(x, 0.0)
    x = jnp.maximum(jnp.dot(x, w2[...], preferred_element_type=f32, precision=jax.lax.Precision.HIGHEST) + b2[...], 0.0)
    x = jnp.maximum(jnp.dot(x, w3[...], preferred_element_type=f32, precision=jax.lax.Precision.HIGHEST) + b3[...], 0.0)
    x = jnp.maximum(jnp.dot(x, w4[...], preferred_element_type=f32, precision=jax.lax.Precision.HIGHEST) + b4[...], 0.0)
    m = jnp.max(x, axis=1, keepdims=True)
    e = jnp.exp(x - m)
    out[...] = e / jnp.sum(e, axis=1, keepdims=True)


def _mlp(movieE, userE, genre2, vote, Wg, bg, W1m, W1u, W1g, w1v, b1,
         W2, b2, W3, b3, W4, b4):
    B = movieE.shape[0]
    T = 2048
    grid = (B // T,)

    def btile(minor):
        return pl.BlockSpec((T, minor), lambda i: (i, 0))

    def full(a):
        return pl.BlockSpec(a.shape, lambda i: (0, 0))

    return pl.pallas_call(
        _mlp_body,
        grid=grid,
        in_specs=[
            btile(movieE.shape[1]),
            btile(userE.shape[1]),
            btile(genre2.shape[1]),
            btile(1),
            full(Wg), full(bg), full(W1m), full(W1u), full(W1g),
            full(w1v), full(b1), full(W2), full(b2), full(W3), full(b3),
            full(W4), full(b4),
        ],
        out_specs=btile(5),
        out_shape=jax.ShapeDtypeStruct((B, 5), jnp.float32),
    )(movieE, userE, genre2, vote, Wg, bg, W1m, W1u, W1g, w1v, b1,
      W2, b2, W3, b3, W4, b4)


def kernel(userId, movieId, genre, vote_average, release_date, movie_table,
           user_table, Wg, bg, W1, b1, W2, b2, W3, b3, W4, b4):
    B = userId.shape[0]
    mids = movieId.reshape(B)
    uids = userId.reshape(B)
    movieE, userE = _sc_gather(movie_table, user_table, mids, uids)
    genre2 = genre.reshape(B, genre.shape[-1])
    W1m = W1[0:20]
    W1u = W1[20:40]
    W1g = W1[40:48]
    w1v = W1[48:49]
    return _mlp(movieE, userE, genre2, vote_average,
                Wg, bg.reshape(1, -1), W1m, W1u, W1g, w1v, b1.reshape(1, -1),
                W2, b2.reshape(1, -1), W3, b3.reshape(1, -1),
                W4, b4.reshape(1, -1))


# speedup vs baseline: 2.0492x; 2.0492x over previous
import functools

import jax
import jax.numpy as jnp
from jax import lax
from jax.experimental import pallas as pl
from jax.experimental.pallas import tpu as pltpu
from jax.experimental.pallas import tpu_sc as plsc

_PAD = 128


def _sc_gather(mt128, ut128, mids, uids):
    B = mids.shape[0]
    info = plsc.get_sparse_core_info()
    nc, ns = info.num_cores, info.num_subcores
    nw = nc * ns
    b_per_w = B // nw
    mesh = plsc.VectorSubcoreMesh(core_axis_name="c", subcore_axis_name="s")

    @functools.partial(
        pl.kernel,
        mesh=mesh,
        compiler_params=pltpu.CompilerParams(use_tc_tiling_on_sc=True),
        out_type=[
            jax.ShapeDtypeStruct((B, _PAD), jnp.float32),
            jax.ShapeDtypeStruct((B, _PAD), jnp.float32),
        ],
        scratch_types=[
            pltpu.VMEM((b_per_w,), jnp.int32),
            pltpu.VMEM((b_per_w,), jnp.int32),
            pltpu.VMEM((b_per_w, _PAD), jnp.float32),
            pltpu.SemaphoreType.DMA,
        ],
    )
    def gather_k(mtab, utab, mid, uid, mout, uout,
                 midx_v, uidx_v, rows_v, sem):
        wid = lax.axis_index("s") * nc + lax.axis_index("c")
        base = wid * b_per_w
        pltpu.sync_copy(mid.at[pl.ds(base, b_per_w)], midx_v)
        pltpu.sync_copy(uid.at[pl.ds(base, b_per_w)], uidx_v)
        pltpu.async_copy(mtab.at[midx_v], rows_v, sem).wait()
        pltpu.sync_copy(rows_v, mout.at[pl.ds(base, b_per_w)])
        pltpu.async_copy(utab.at[uidx_v], rows_v, sem).wait()
        pltpu.sync_copy(rows_v, uout.at[pl.ds(base, b_per_w)])

    return gather_k(mt128, ut128, mids, uids)


def _mlp_body(mv, us, gn, vt, wg, bg, w1m, w1u, w1g, w1v, b1,
              w2, b2, w3, b3, w4, b4, out):
    f32 = jnp.float32
    emb = w1m.shape[0]
    g = jnp.dot(gn[...], wg[...], preferred_element_type=f32) + bg[...]
    g = jnp.maximum(g, 0.0)
    x = (jnp.dot(mv[:, 0:emb], w1m[...], preferred_element_type=f32)
         + jnp.dot(us[:, 0:emb], w1u[...], preferred_element_type=f32)
         + jnp.dot(g, w1g[...], preferred_element_type=f32)
         + vt[...] * w1v[...]
         + b1[...])
    x = jnp.maximum(x, 0.0)
    x = jnp.maximum(jnp.dot(x, w2[...], preferred_element_type=f32) + b2[...], 0.0)
    x = jnp.maximum(jnp.dot(x, w3[...], preferred_element_type=f32) + b3[...], 0.0)
    x = jnp.maximum(jnp.dot(x, w4[...], preferred_element_type=f32) + b4[...], 0.0)
    m = jnp.max(x, axis=1, keepdims=True)
    e = jnp.exp(x - m)
    out[...] = e / jnp.sum(e, axis=1, keepdims=True)


def _mlp(movieE, userE, genre2, vote, Wg, bg, W1m, W1u, W1g, w1v, b1,
         W2, b2, W3, b3, W4, b4):
    B = movieE.shape[0]
    T = 2048
    grid = (B // T,)

    def btile(minor):
        return pl.BlockSpec((T, minor), lambda i: (i, 0))

    def full(a):
        return pl.BlockSpec(a.shape, lambda i: (0, 0))

    return pl.pallas_call(
        _mlp_body,
        grid=grid,
        in_specs=[
            btile(movieE.shape[1]),
            btile(userE.shape[1]),
            btile(genre2.shape[1]),
            btile(1),
            full(Wg), full(bg), full(W1m), full(W1u), full(W1g),
            full(w1v), full(b1), full(W2), full(b2), full(W3), full(b3),
            full(W4), full(b4),
        ],
        out_specs=btile(5),
        out_shape=jax.ShapeDtypeStruct((B, 5), jnp.float32),
    )(movieE, userE, genre2, vote, Wg, bg, W1m, W1u, W1g, w1v, b1,
      W2, b2, W3, b3, W4, b4)


def kernel(userId, movieId, genre, vote_average, release_date, movie_table,
           user_table, Wg, bg, W1, b1, W2, b2, W3, b3, W4, b4):
    B = userId.shape[0]
    emb = movie_table.shape[1]
    mids = movieId.reshape(B)
    uids = userId.reshape(B)
    pad = ((0, 0), (0, _PAD - emb))
    mt128 = jnp.pad(movie_table, pad)
    ut128 = jnp.pad(user_table, pad)
    movieE, userE = _sc_gather(mt128, ut128, mids, uids)
    genre2 = genre.reshape(B, genre.shape[-1])
    W1m = W1[0:20]
    W1u = W1[20:40]
    W1g = W1[40:48]
    w1v = W1[48:49]
    return _mlp(movieE, userE, genre2, vote_average,
                Wg, bg.reshape(1, -1), W1m, W1u, W1g, w1v, b1.reshape(1, -1),
                W2, b2.reshape(1, -1), W3, b3.reshape(1, -1),
                W4, b4.reshape(1, -1))


# speedup vs baseline: 2.8153x; 1.3739x over previous
import functools

import jax
import jax.numpy as jnp
from jax import lax
from jax.experimental import pallas as pl
from jax.experimental.pallas import tpu as pltpu
from jax.experimental.pallas import tpu_sc as plsc

_PAD = 128


def _sc_gather(mt128, ut128, mids, uids):
    B = mids.shape[0]
    info = plsc.get_sparse_core_info()
    nc, ns = info.num_cores, info.num_subcores
    nw = nc * ns
    b_per_w = B // nw
    mesh = plsc.VectorSubcoreMesh(core_axis_name="c", subcore_axis_name="s")

    @functools.partial(
        pl.kernel,
        mesh=mesh,
        compiler_params=pltpu.CompilerParams(use_tc_tiling_on_sc=True),
        out_type=[
            jax.ShapeDtypeStruct((B, _PAD), jnp.float32),
            jax.ShapeDtypeStruct((B, _PAD), jnp.float32),
        ],
        scratch_types=[
            pltpu.VMEM((b_per_w,), jnp.int32),
            pltpu.VMEM((b_per_w,), jnp.int32),
            pltpu.VMEM((b_per_w, _PAD), jnp.float32),
            pltpu.SemaphoreType.DMA,
        ],
    )
    def gather_k(mtab, utab, mid, uid, mout, uout,
                 midx_v, uidx_v, rows_v, sem):
        wid = lax.axis_index("s") * nc + lax.axis_index("c")
        base = wid * b_per_w
        pltpu.sync_copy(mid.at[pl.ds(base, b_per_w)], midx_v)
        pltpu.sync_copy(uid.at[pl.ds(base, b_per_w)], uidx_v)
        pltpu.async_copy(mtab.at[midx_v], rows_v, sem).wait()
        pltpu.sync_copy(rows_v, mout.at[pl.ds(base, b_per_w)])
        pltpu.async_copy(utab.at[uidx_v], rows_v, sem).wait()
        pltpu.sync_copy(rows_v, uout.at[pl.ds(base, b_per_w)])

    return gather_k(mt128, ut128, mids, uids)


def _prep_body(mT, uT, eye, mo, uo):
    f32 = jnp.float32
    dims = (((0,), (0,)), ((), ()))
    mo[...] = jax.lax.dot_general(mT[...], eye[...], dims,
                                  preferred_element_type=f32)
    uo[...] = jax.lax.dot_general(uT[...], eye[...], dims,
                                  preferred_element_type=f32)


def _prep(movieT, userT, eye):
    V = movieT.shape[1]
    C = 4096
    grid = ((V + C - 1) // C,)
    return pl.pallas_call(
        _prep_body,
        grid=grid,
        in_specs=[
            pl.BlockSpec((movieT.shape[0], C), lambda i: (0, i)),
            pl.BlockSpec((userT.shape[0], C), lambda i: (0, i)),
            pl.BlockSpec(eye.shape, lambda i: (0, 0)),
        ],
        out_specs=[
            pl.BlockSpec((C, _PAD), lambda i: (i, 0)),
            pl.BlockSpec((C, _PAD), lambda i: (i, 0)),
        ],
        out_shape=[
            jax.ShapeDtypeStruct((V, _PAD), jnp.float32),
            jax.ShapeDtypeStruct((V, _PAD), jnp.float32),
        ],
    )(movieT, userT, eye)


def _mlp_body(mv, us, gn, vt, wg, bg, w1m, w1u, w1g, w1v, b1,
              w2, b2, w3, b3, w4, b4, out):
    f32 = jnp.float32
    emb = w1m.shape[0]
    g = jnp.dot(gn[...], wg[...], preferred_element_type=f32) + bg[...]
    g = jnp.maximum(g, 0.0)
    x = (jnp.dot(mv[:, 0:emb], w1m[...], preferred_element_type=f32)
         + jnp.dot(us[:, 0:emb], w1u[...], preferred_element_type=f32)
         + jnp.dot(g, w1g[...], preferred_element_type=f32)
         + vt[...] * w1v[...]
         + b1[...])
    x = jnp.maximum(x, 0.0)
    x = jnp.maximum(jnp.dot(x, w2[...], preferred_element_type=f32) + b2[...], 0.0)
    x = jnp.maximum(jnp.dot(x, w3[...], preferred_element_type=f32) + b3[...], 0.0)
    x = jnp.maximum(jnp.dot(x, w4[...], preferred_element_type=f32) + b4[...], 0.0)
    m = jnp.max(x, axis=1, keepdims=True)
    e = jnp.exp(x - m)
    out[...] = e / jnp.sum(e, axis=1, keepdims=True)


def _mlp(movieE, userE, genre2, vote, Wg, bg, W1m, W1u, W1g, w1v, b1,
         W2, b2, W3, b3, W4, b4):
    B = movieE.shape[0]
    T = 2048
    grid = (B // T,)

    def btile(minor):
        return pl.BlockSpec((T, minor), lambda i: (i, 0))

    def full(a):
        return pl.BlockSpec(a.shape, lambda i: (0, 0))

    return pl.pallas_call(
        _mlp_body,
        grid=grid,
        in_specs=[
            btile(movieE.shape[1]),
            btile(userE.shape[1]),
            btile(genre2.shape[1]),
            btile(1),
            full(Wg), full(bg), full(W1m), full(W1u), full(W1g),
            full(w1v), full(b1), full(W2), full(b2), full(W3), full(b3),
            full(W4), full(b4),
        ],
        out_specs=btile(5),
        out_shape=jax.ShapeDtypeStruct((B, 5), jnp.float32),
    )(movieE, userE, genre2, vote, Wg, bg, W1m, W1u, W1g, w1v, b1,
      W2, b2, W3, b3, W4, b4)


def kernel(userId, movieId, genre, vote_average, release_date, movie_table,
           user_table, Wg, bg, W1, b1, W2, b2, W3, b3, W4, b4):
    B = userId.shape[0]
    emb = movie_table.shape[1]
    mids = movieId.reshape(B)
    uids = userId.reshape(B)
    eye = jnp.eye(emb, _PAD, dtype=jnp.float32)
    mt128, ut128 = _prep(movie_table.T, user_table.T, eye)
    movieE, userE = _sc_gather(mt128, ut128, mids, uids)
    genre2 = genre.reshape(B, genre.shape[-1])
    W1m = W1[0:20]
    W1u = W1[20:40]
    W1g = W1[40:48]
    w1v = W1[48:49]
    return _mlp(movieE, userE, genre2, vote_average,
                Wg, bg.reshape(1, -1), W1m, W1u, W1g, w1v, b1.reshape(1, -1),
                W2, b2.reshape(1, -1), W3, b3.reshape(1, -1),
                W4, b4.reshape(1, -1))


# speedup vs baseline: 2.8793x; 1.0227x over previous
import functools

import jax
import jax.numpy as jnp
from jax import lax
from jax.experimental import pallas as pl
from jax.experimental.pallas import tpu as pltpu
from jax.experimental.pallas import tpu_sc as plsc

_PAD = 128


def _sc_gather(mt128, ut128, mids, uids):
    B = mids.shape[0]
    info = plsc.get_sparse_core_info()
    nc, ns = info.num_cores, info.num_subcores
    nw = nc * ns
    b_per_w = B // nw
    mesh = plsc.VectorSubcoreMesh(core_axis_name="c", subcore_axis_name="s")

    @functools.partial(
        pl.kernel,
        mesh=mesh,
        compiler_params=pltpu.CompilerParams(use_tc_tiling_on_sc=True),
        out_type=[
            jax.ShapeDtypeStruct((B, _PAD), jnp.float32),
            jax.ShapeDtypeStruct((B, _PAD), jnp.float32),
        ],
        scratch_types=[
            pltpu.VMEM((b_per_w,), jnp.int32),
            pltpu.VMEM((b_per_w,), jnp.int32),
            pltpu.VMEM((b_per_w // 2, _PAD), jnp.float32),
            pltpu.VMEM((b_per_w // 2, _PAD), jnp.float32),
            pltpu.SemaphoreType.DMA,
            pltpu.SemaphoreType.DMA,
        ],
    )
    def gather_k(mtab, utab, mid, uid, mout, uout,
                 midx_v, uidx_v, buf_a, buf_b, sem_a, sem_b):
        wid = lax.axis_index("s") * nc + lax.axis_index("c")
        base = wid * b_per_w
        half = b_per_w // 2
        pltpu.sync_copy(mid.at[pl.ds(base, b_per_w)], midx_v)
        pltpu.sync_copy(uid.at[pl.ds(base, b_per_w)], uidx_v)
        ma = pltpu.async_copy(mtab.at[midx_v.at[pl.ds(0, half)]], buf_a, sem_a)
        mb = pltpu.async_copy(mtab.at[midx_v.at[pl.ds(half, half)]], buf_b, sem_b)
        ma.wait()
        pltpu.sync_copy(buf_a, mout.at[pl.ds(base, half)])
        ua = pltpu.async_copy(utab.at[uidx_v.at[pl.ds(0, half)]], buf_a, sem_a)
        mb.wait()
        pltpu.sync_copy(buf_b, mout.at[pl.ds(base + half, half)])
        ub = pltpu.async_copy(utab.at[uidx_v.at[pl.ds(half, half)]], buf_b, sem_b)
        ua.wait()
        pltpu.sync_copy(buf_a, uout.at[pl.ds(base, half)])
        ub.wait()
        pltpu.sync_copy(buf_b, uout.at[pl.ds(base + half, half)])

    return gather_k(mt128, ut128, mids, uids)


def _prep_body(mT, uT, eye, mo, uo):
    f32 = jnp.float32
    dims = (((0,), (0,)), ((), ()))
    mo[...] = jax.lax.dot_general(mT[...], eye[...], dims,
                                  preferred_element_type=f32)
    uo[...] = jax.lax.dot_general(uT[...], eye[...], dims,
                                  preferred_element_type=f32)


def _prep(movieT, userT, eye):
    V = movieT.shape[1]
    C = 4096
    grid = ((V + C - 1) // C,)
    return pl.pallas_call(
        _prep_body,
        grid=grid,
        in_specs=[
            pl.BlockSpec((movieT.shape[0], C), lambda i: (0, i)),
            pl.BlockSpec((userT.shape[0], C), lambda i: (0, i)),
            pl.BlockSpec(eye.shape, lambda i: (0, 0)),
        ],
        out_specs=[
            pl.BlockSpec((C, _PAD), lambda i: (i, 0)),
            pl.BlockSpec((C, _PAD), lambda i: (i, 0)),
        ],
        out_shape=[
            jax.ShapeDtypeStruct((V, _PAD), jnp.float32),
            jax.ShapeDtypeStruct((V, _PAD), jnp.float32),
        ],
    )(movieT, userT, eye)


def _mlp_body(mv, us, gnT, vt, wg, bg, w1m, w1u, w1g, w1v, b1,
              w2, b2, w3, b3, w4, b4, out):
    f32 = jnp.float32
    emb = w1m.shape[0]
    dims = (((0,), (0,)), ((), ()))
    g = jax.lax.dot_general(gnT[...], wg[...], dims,
                            preferred_element_type=f32) + bg[...]
    g = jnp.maximum(g, 0.0)
    x = (jnp.dot(mv[:, 0:emb], w1m[...], preferred_element_type=f32)
         + jnp.dot(us[:, 0:emb], w1u[...], preferred_element_type=f32)
         + jnp.dot(g, w1g[...], preferred_element_type=f32)
         + vt[...] * w1v[...]
         + b1[...])
    x = jnp.maximum(x, 0.0)
    x = jnp.maximum(jnp.dot(x, w2[...], preferred_element_type=f32) + b2[...], 0.0)
    x = jnp.maximum(jnp.dot(x, w3[...], preferred_element_type=f32) + b3[...], 0.0)
    x = jnp.maximum(jnp.dot(x, w4[...], preferred_element_type=f32) + b4[...], 0.0)
    m = jnp.max(x, axis=1, keepdims=True)
    e = jnp.exp(x - m)
    out[...] = e / jnp.sum(e, axis=1, keepdims=True)


def _mlp(movieE, userE, genreT, vote, Wg, bg, W1m, W1u, W1g, w1v, b1,
         W2, b2, W3, b3, W4, b4):
    B = movieE.shape[0]
    T = 2048
    grid = (B // T,)

    def btile(minor):
        return pl.BlockSpec((T, minor), lambda i: (i, 0))

    def full(a):
        return pl.BlockSpec(a.shape, lambda i: (0, 0))

    return pl.pallas_call(
        _mlp_body,
        grid=grid,
        in_specs=[
            btile(movieE.shape[1]),
            btile(userE.shape[1]),
            pl.BlockSpec((genreT.shape[0], T), lambda i: (0, i)),
            btile(1),
            full(Wg), full(bg), full(W1m), full(W1u), full(W1g),
            full(w1v), full(b1), full(W2), full(b2), full(W3), full(b3),
            full(W4), full(b4),
        ],
        out_specs=btile(5),
        out_shape=jax.ShapeDtypeStruct((B, 5), jnp.float32),
    )(movieE, userE, genreT, vote, Wg, bg, W1m, W1u, W1g, w1v, b1,
      W2, b2, W3, b3, W4, b4)


def kernel(userId, movieId, genre, vote_average, release_date, movie_table,
           user_table, Wg, bg, W1, b1, W2, b2, W3, b3, W4, b4):
    B = userId.shape[0]
    emb = movie_table.shape[1]
    mids = movieId.reshape(B)
    uids = userId.reshape(B)
    eye = jnp.eye(emb, _PAD, dtype=jnp.float32)
    mt128, ut128 = _prep(movie_table.T, user_table.T, eye)
    movieE, userE = _sc_gather(mt128, ut128, mids, uids)
    genreT = genre.reshape(B, genre.shape[-1]).T
    W1m = W1[0:20]
    W1u = W1[20:40]
    W1g = W1[40:48]
    w1v = W1[48:49]
    return _mlp(movieE, userE, genreT, vote_average,
                Wg, bg.reshape(1, -1), W1m, W1u, W1g, w1v, b1.reshape(1, -1),
                W2, b2.reshape(1, -1), W3, b3.reshape(1, -1),
                W4, b4.reshape(1, -1))
